# Initial kernel scaffold; baseline (speedup 1.0000x reference)
import jax
import jax.numpy as jnp
from jax.experimental import pallas as pl


def kernel(utter_hidden, edge_index, edge_weight, posemb, W1, b1, Wl, bl):
    raise NotImplementedError("write your pallas kernel here")


# speedup vs baseline: 12.1467x; 12.1467x over previous
import functools

import jax
import jax.numpy as jnp
from jax import lax
from jax.experimental import pallas as pl
from jax.experimental.pallas import tpu as pltpu
from jax.experimental.pallas import tpu_sc as plsc

FEAT = 128
N_EDGES = 320000
NP = 10240
CH = 80

_info = plsc.get_sparse_core_info()
NC = _info.num_cores
NS = _info.num_subcores
NW = NC * NS
EPW = N_EDGES // NW
CHUNKS = EPW // CH
BLK = 25
NBLK = CHUNKS // BLK
RPT = NP // NS
ZR = 128

_MESH = plsc.VectorSubcoreMesh(core_axis_name="c", subcore_axis_name="s")


@functools.partial(
    pl.kernel,
    mesh=_MESH,
    out_type=jax.ShapeDtypeStruct((NC, NP), jnp.float32),
    scratch_types=[
        pltpu.VMEM((CHUNKS, CH), jnp.int32),
        pltpu.VMEM((CHUNKS, CH), jnp.float32),
        pltpu.VMEM((RPT,), jnp.float32),
        pltpu.VMEM_SHARED((NP,), jnp.float32),
    ],
)
def _deg_kernel(col_hbm, ew_hbm, out_hbm, col_v, ew_v, zbuf, deg_sh):
    c = lax.axis_index("c")
    s = lax.axis_index("s")
    wid = s * NC + c
    zero16 = jnp.zeros((16,), jnp.float32)

    def zinit(j, _):
        zbuf[pl.ds(j * 16, 16)] = zero16
        return 0

    lax.fori_loop(0, RPT // 16, zinit, 0)
    pltpu.sync_copy(zbuf, deg_sh.at[pl.ds(s * RPT, RPT)])
    pltpu.sync_copy(col_hbm.at[wid], col_v)
    pltpu.sync_copy(ew_hbm.at[wid], ew_v)
    plsc.subcore_barrier()

    def chunk(i, _):
        pltpu.sync_copy(ew_v.at[i], deg_sh.at[col_v.at[i]], add=True)
        return 0

    lax.fori_loop(0, CHUNKS, chunk, 0)
    plsc.subcore_barrier()
    pltpu.sync_copy(deg_sh.at[pl.ds(s * RPT, RPT)],
                    out_hbm.at[c, pl.ds(s * RPT, RPT)])


@functools.partial(
    pl.kernel,
    mesh=_MESH,
    out_type=jax.ShapeDtypeStruct((NC, NP, FEAT), jnp.float32),
    scratch_types=[
        pltpu.VMEM((BLK, CH), jnp.int32),
        pltpu.VMEM((BLK, CH), jnp.int32),
        pltpu.VMEM((BLK, CH), jnp.float32),
        pltpu.VMEM((CH, FEAT), jnp.float32),
        pltpu.VMEM_SHARED((NP, FEAT), jnp.float32),
        pltpu.SemaphoreType.DMA,
    ],
)
def _spmm_kernel(g_hbm, row_hbm, col_hbm, ew_hbm, out_hbm,
                 row_v, col_v, ew_v, rows_v, acc_sh, sem):
    c = lax.axis_index("c")
    s = lax.axis_index("s")
    wid = s * NC + c
    zero16 = jnp.zeros((16,), jnp.float32)

    def zinit(j, _):
        for d in range(FEAT // 16):
            rows_v[j, pl.ds(d * 16, 16)] = zero16
        return 0

    lax.fori_loop(0, CH, zinit, 0)
    for k in range(RPT // CH):
        pltpu.sync_copy(rows_v, acc_sh.at[pl.ds(s * RPT + k * CH, CH)])
    plsc.subcore_barrier()

    def blk_body(bi, _):
        pltpu.sync_copy(row_hbm.at[wid, bi], row_v)
        pltpu.sync_copy(col_hbm.at[wid, bi], col_v)
        pltpu.sync_copy(ew_hbm.at[wid, bi], ew_v)

        def chunk(i, _):
            pltpu.async_copy(g_hbm.at[row_v.at[i]], rows_v, sem).wait()

            def scale(gi, _):
                wv = ew_v[i, pl.ds(gi * 16, 16)]
                for l in range(16):
                    w = wv[l]
                    j = gi * 16 + l
                    for d in range(FEAT // 16):
                        sl = pl.ds(d * 16, 16)
                        rows_v[j, sl] = rows_v[j, sl] * w
                return 0

            lax.fori_loop(0, CH // 16, scale, 0)
            pltpu.sync_copy(rows_v, acc_sh.at[col_v.at[i]], add=True)
            return 0

        lax.fori_loop(0, BLK, chunk, 0)
        return 0

    lax.fori_loop(0, NBLK, blk_body, 0)
    plsc.subcore_barrier()
    pltpu.sync_copy(acc_sh.at[pl.ds(s * RPT, RPT)],
                    out_hbm.at[c, pl.ds(s * RPT, RPT)])


_GRID = NP // 512


def _bs2(r, c_, im):
    return pl.BlockSpec((r, c_), im)


def _prep_body(part_ref, x_ref, w_ref, h_ref, g_ref, dis_ref, inv_ref):
    deg = part_ref[0, :] + part_ref[1, :] + 1.0
    dis = lax.rsqrt(deg)
    inv = 1.0 / deg
    h = jnp.dot(x_ref[...], w_ref[...], preferred_element_type=jnp.float32)
    h_ref[...] = h
    g_ref[...] = h * dis[:, None]
    dis_ref[...] = dis[:, None]
    inv_ref[...] = inv[:, None]


def _tc_prep(parts, x, W1):
    return pl.pallas_call(
        _prep_body,
        grid=(_GRID,),
        in_specs=[
            _bs2(2, 512, lambda i: (0, i)),
            _bs2(512, FEAT, lambda i: (i, 0)),
            _bs2(FEAT, FEAT, lambda i: (0, 0)),
        ],
        out_specs=[
            _bs2(512, FEAT, lambda i: (i, 0)),
            _bs2(512, FEAT, lambda i: (i, 0)),
            _bs2(512, 1, lambda i: (i, 0)),
            _bs2(512, 1, lambda i: (i, 0)),
        ],
        out_shape=[
            jax.ShapeDtypeStruct((NP, FEAT), jnp.float32),
            jax.ShapeDtypeStruct((NP, FEAT), jnp.float32),
            jax.ShapeDtypeStruct((NP, 1), jnp.float32),
            jax.ShapeDtypeStruct((NP, 1), jnp.float32),
        ],
    )(parts, x, W1)


def _mid_body(sp_ref, h_ref, dis_ref, inv_ref, b1_ref, w_ref,
              x_ref, hn_ref, gn_ref):
    sacc = sp_ref[0] + sp_ref[1]
    xl = jnp.maximum(
        dis_ref[...] * sacc + inv_ref[...] * h_ref[...] + b1_ref[...], 0.0)
    x_ref[...] = xl
    hn = jnp.dot(xl, w_ref[...], preferred_element_type=jnp.float32)
    hn_ref[...] = hn
    gn_ref[...] = dis_ref[...] * hn


def _tc_mid(sp, h, dis, inv, b1r, W1):
    return pl.pallas_call(
        _mid_body,
        grid=(_GRID,),
        in_specs=[
            pl.BlockSpec((2, 512, FEAT), lambda i: (0, i, 0)),
            _bs2(512, FEAT, lambda i: (i, 0)),
            _bs2(512, 1, lambda i: (i, 0)),
            _bs2(512, 1, lambda i: (i, 0)),
            _bs2(1, FEAT, lambda i: (0, 0)),
            _bs2(FEAT, FEAT, lambda i: (0, 0)),
        ],
        out_specs=[
            _bs2(512, FEAT, lambda i: (i, 0)),
            _bs2(512, FEAT, lambda i: (i, 0)),
            _bs2(512, FEAT, lambda i: (i, 0)),
        ],
        out_shape=[
            jax.ShapeDtypeStruct((NP, FEAT), jnp.float32),
            jax.ShapeDtypeStruct((NP, FEAT), jnp.float32),
            jax.ShapeDtypeStruct((NP, FEAT), jnp.float32),
        ],
    )(sp, h, dis, inv, b1r, W1)


def _fin_body(sp_ref, h_ref, dis_ref, inv_ref, b1_ref, x1_ref, x2_ref,
              wl_ref, bl_ref, y_ref):
    sacc = sp_ref[0] + sp_ref[1]
    x3 = jnp.maximum(
        dis_ref[...] * sacc + inv_ref[...] * h_ref[...] + b1_ref[...], 0.0)
    xs = x1_ref[...] + x2_ref[...] + x3
    y_ref[...] = jnp.tanh(
        jnp.dot(xs, wl_ref[...], preferred_element_type=jnp.float32)
        + bl_ref[...])


def _tc_fin(sp, h, dis, inv, b1r, x1, x2, Wl, blr):
    return pl.pallas_call(
        _fin_body,
        grid=(_GRID,),
        in_specs=[
            pl.BlockSpec((2, 512, FEAT), lambda i: (0, i, 0)),
            _bs2(512, FEAT, lambda i: (i, 0)),
            _bs2(512, 1, lambda i: (i, 0)),
            _bs2(512, 1, lambda i: (i, 0)),
            _bs2(1, FEAT, lambda i: (0, 0)),
            _bs2(512, FEAT, lambda i: (i, 0)),
            _bs2(512, FEAT, lambda i: (i, 0)),
            _bs2(FEAT, FEAT, lambda i: (0, 0)),
            _bs2(1, FEAT, lambda i: (0, 0)),
        ],
        out_specs=_bs2(512, FEAT, lambda i: (i, 0)),
        out_shape=jax.ShapeDtypeStruct((NP, FEAT), jnp.float32),
    )(sp, h, dis, inv, b1r, x1, x2, Wl, blr)


def kernel(utter_hidden, edge_index, edge_weight, posemb, W1, b1, Wl, bl):
    turn, batch, _ = utter_hidden.shape
    n = turn * batch
    x = jnp.transpose(utter_hidden, (1, 0, 2)).reshape(n, -1)
    pe = jnp.tile(posemb[:turn], (batch, 1))
    x = jnp.concatenate([x, pe], axis=1)
    x = jnp.zeros((NP, FEAT), jnp.float32).at[:n].set(x)

    row4 = edge_index[0].reshape(NW, NBLK, BLK, CH)
    col4 = edge_index[1].reshape(NW, NBLK, BLK, CH)
    ew4 = edge_weight.reshape(NW, NBLK, BLK, CH)
    col2 = edge_index[1].reshape(NW, CHUNKS, CH)
    ew2 = edge_weight.reshape(NW, CHUNKS, CH)
    b1r = b1.reshape(1, -1)
    blr = bl.reshape(1, -1)

    parts = _deg_kernel(col2, ew2)
    h1, g1, dis, inv = _tc_prep(parts, x, W1)
    s1 = _spmm_kernel(g1, row4, col4, ew4)
    x1, h2, g2 = _tc_mid(s1, h1, dis, inv, b1r, W1)
    s2 = _spmm_kernel(g2, row4, col4, ew4)
    x2, h3, g3 = _tc_mid(s2, h2, dis, inv, b1r, W1)
    s3 = _spmm_kernel(g3, row4, col4, ew4)
    y = _tc_fin(s3, h3, dis, inv, b1r, x1, x2, Wl, blr)
    return y[:n].reshape(batch, turn, -1)


# speedup vs baseline: 17.5157x; 1.4420x over previous
import functools

import jax
import jax.numpy as jnp
from jax import lax
from jax.experimental import pallas as pl
from jax.experimental.pallas import tpu as pltpu
from jax.experimental.pallas import tpu_sc as plsc

FEAT = 128
N_EDGES = 320000
NP = 10240
CH = 80

_info = plsc.get_sparse_core_info()
NC = _info.num_cores
NS = _info.num_subcores
NW = NC * NS
EPW = N_EDGES // NW
CHUNKS = EPW // CH
BLK = 25
NBLK = CHUNKS // BLK
RPT = NP // NS
ZR = 128

_MESH = plsc.VectorSubcoreMesh(core_axis_name="c", subcore_axis_name="s")


@functools.partial(
    pl.kernel,
    mesh=_MESH,
    out_type=jax.ShapeDtypeStruct((NC, NP), jnp.float32),
    scratch_types=[
        pltpu.VMEM((CHUNKS, CH), jnp.int32),
        pltpu.VMEM((CHUNKS, CH), jnp.float32),
        pltpu.VMEM((RPT,), jnp.float32),
        pltpu.VMEM_SHARED((NP,), jnp.float32),
    ],
)
def _deg_kernel(col_hbm, ew_hbm, out_hbm, col_v, ew_v, zbuf, deg_sh):
    c = lax.axis_index("c")
    s = lax.axis_index("s")
    wid = s * NC + c
    zero16 = jnp.zeros((16,), jnp.float32)

    def zinit(j, _):
        zbuf[pl.ds(j * 16, 16)] = zero16
        return 0

    lax.fori_loop(0, RPT // 16, zinit, 0)
    pltpu.sync_copy(zbuf, deg_sh.at[pl.ds(s * RPT, RPT)])
    pltpu.sync_copy(col_hbm.at[wid], col_v)
    pltpu.sync_copy(ew_hbm.at[wid], ew_v)
    plsc.subcore_barrier()

    def chunk(i, _):
        pltpu.sync_copy(ew_v.at[i], deg_sh.at[col_v.at[i]], add=True)
        return 0

    lax.fori_loop(0, CHUNKS, chunk, 0)
    plsc.subcore_barrier()
    pltpu.sync_copy(deg_sh.at[pl.ds(s * RPT, RPT)],
                    out_hbm.at[c, pl.ds(s * RPT, RPT)])


@functools.partial(
    pl.kernel,
    mesh=_MESH,
    out_type=jax.ShapeDtypeStruct((NC, NP, FEAT), jnp.float32),
    scratch_types=[
        pltpu.VMEM((BLK, CH), jnp.int32),
        pltpu.VMEM((BLK, CH), jnp.int32),
        pltpu.VMEM((BLK, CH), jnp.float32),
        pltpu.VMEM((CH, FEAT), jnp.float32),
        pltpu.VMEM((CH, FEAT), jnp.float32),
        pltpu.VMEM_SHARED((NP, FEAT), jnp.float32),
        pltpu.SemaphoreType.DMA,
        pltpu.SemaphoreType.DMA,
    ],
)
def _spmm_kernel(g_hbm, row_hbm, col_hbm, ew_hbm, out_hbm,
                 row_v, col_v, ew_v, rows_a, rows_b, acc_sh, semg0, semg1):
    c = lax.axis_index("c")
    s = lax.axis_index("s")
    wid = s * NC + c
    zero16 = jnp.zeros((16,), jnp.float32)

    def zinit(j, _):
        for d in range(FEAT // 16):
            rows_a[j, pl.ds(d * 16, 16)] = zero16
        return 0

    lax.fori_loop(0, CH, zinit, 0)
    for k in range(RPT // CH):
        pltpu.sync_copy(rows_a, acc_sh.at[pl.ds(s * RPT + k * CH, CH)])
    plsc.subcore_barrier()

    def _scale(buf, i):
        def grp(gi, _):
            wv = ew_v[i, pl.ds(gi * 16, 16)]
            for l in range(16):
                w = wv[l]
                j = gi * 16 + l
                for d in range(FEAT // 16):
                    sl = pl.ds(d * 16, 16)
                    buf[j, sl] = buf[j, sl] * w
            return 0

        lax.fori_loop(0, CH // 16, grp, 0)

    def blk_body(bi, _):
        pltpu.sync_copy(row_hbm.at[wid, bi], row_v)
        pltpu.sync_copy(col_hbm.at[wid, bi], col_v)
        pltpu.sync_copy(ew_hbm.at[wid, bi], ew_v)
        pltpu.async_copy(g_hbm.at[row_v.at[0]], rows_a, semg0)

        def pair(k, _):
            i0 = 2 * k
            i1 = 2 * k + 1
            i2 = 2 * k + 2
            pltpu.make_async_copy(g_hbm.at[row_v.at[i0]], rows_a, semg0).wait()
            pltpu.async_copy(g_hbm.at[row_v.at[i1]], rows_b, semg1)
            _scale(rows_a, i0)
            pltpu.sync_copy(rows_a, acc_sh.at[col_v.at[i0]], add=True)
            pltpu.make_async_copy(g_hbm.at[row_v.at[i1]], rows_b, semg1).wait()
            pltpu.async_copy(g_hbm.at[row_v.at[i2]], rows_a, semg0)
            _scale(rows_b, i1)
            pltpu.sync_copy(rows_b, acc_sh.at[col_v.at[i1]], add=True)
            return 0

        lax.fori_loop(0, BLK // 2, pair, 0)
        tail = BLK - 1
        pltpu.make_async_copy(g_hbm.at[row_v.at[tail]], rows_a, semg0).wait()
        _scale(rows_a, tail)
        pltpu.sync_copy(rows_a, acc_sh.at[col_v.at[tail]], add=True)
        return 0

    lax.fori_loop(0, NBLK, blk_body, 0)
    plsc.subcore_barrier()
    pltpu.sync_copy(acc_sh.at[pl.ds(s * RPT, RPT)],
                    out_hbm.at[c, pl.ds(s * RPT, RPT)])


_GRID = NP // 512


def _bs2(r, c_, im):
    return pl.BlockSpec((r, c_), im)


def _prep_body(part_ref, x_ref, w_ref, h_ref, g_ref, dis_ref, inv_ref):
    deg = part_ref[0, :] + part_ref[1, :] + 1.0
    dis = lax.rsqrt(deg)
    inv = 1.0 / deg
    h = jnp.dot(x_ref[...], w_ref[...], preferred_element_type=jnp.float32)
    h_ref[...] = h
    g_ref[...] = h * dis[:, None]
    dis_ref[...] = dis[:, None]
    inv_ref[...] = inv[:, None]


def _tc_prep(parts, x, W1):
    return pl.pallas_call(
        _prep_body,
        grid=(_GRID,),
        in_specs=[
            _bs2(2, 512, lambda i: (0, i)),
            _bs2(512, FEAT, lambda i: (i, 0)),
            _bs2(FEAT, FEAT, lambda i: (0, 0)),
        ],
        out_specs=[
            _bs2(512, FEAT, lambda i: (i, 0)),
            _bs2(512, FEAT, lambda i: (i, 0)),
            _bs2(512, 1, lambda i: (i, 0)),
            _bs2(512, 1, lambda i: (i, 0)),
        ],
        out_shape=[
            jax.ShapeDtypeStruct((NP, FEAT), jnp.float32),
            jax.ShapeDtypeStruct((NP, FEAT), jnp.float32),
            jax.ShapeDtypeStruct((NP, 1), jnp.float32),
            jax.ShapeDtypeStruct((NP, 1), jnp.float32),
        ],
    )(parts, x, W1)


def _mid_body(sp_ref, h_ref, dis_ref, inv_ref, b1_ref, w_ref,
              x_ref, hn_ref, gn_ref):
    sacc = sp_ref[0] + sp_ref[1]
    xl = jnp.maximum(
        dis_ref[...] * sacc + inv_ref[...] * h_ref[...] + b1_ref[...], 0.0)
    x_ref[...] = xl
    hn = jnp.dot(xl, w_ref[...], preferred_element_type=jnp.float32)
    hn_ref[...] = hn
    gn_ref[...] = dis_ref[...] * hn


def _tc_mid(sp, h, dis, inv, b1r, W1):
    return pl.pallas_call(
        _mid_body,
        grid=(_GRID,),
        in_specs=[
            pl.BlockSpec((2, 512, FEAT), lambda i: (0, i, 0)),
            _bs2(512, FEAT, lambda i: (i, 0)),
            _bs2(512, 1, lambda i: (i, 0)),
            _bs2(512, 1, lambda i: (i, 0)),
            _bs2(1, FEAT, lambda i: (0, 0)),
            _bs2(FEAT, FEAT, lambda i: (0, 0)),
        ],
        out_specs=[
            _bs2(512, FEAT, lambda i: (i, 0)),
            _bs2(512, FEAT, lambda i: (i, 0)),
            _bs2(512, FEAT, lambda i: (i, 0)),
        ],
        out_shape=[
            jax.ShapeDtypeStruct((NP, FEAT), jnp.float32),
            jax.ShapeDtypeStruct((NP, FEAT), jnp.float32),
            jax.ShapeDtypeStruct((NP, FEAT), jnp.float32),
        ],
    )(sp, h, dis, inv, b1r, W1)


def _fin_body(sp_ref, h_ref, dis_ref, inv_ref, b1_ref, x1_ref, x2_ref,
              wl_ref, bl_ref, y_ref):
    sacc = sp_ref[0] + sp_ref[1]
    x3 = jnp.maximum(
        dis_ref[...] * sacc + inv_ref[...] * h_ref[...] + b1_ref[...], 0.0)
    xs = x1_ref[...] + x2_ref[...] + x3
    y_ref[...] = jnp.tanh(
        jnp.dot(xs, wl_ref[...], preferred_element_type=jnp.float32)
        + bl_ref[...])


def _tc_fin(sp, h, dis, inv, b1r, x1, x2, Wl, blr):
    return pl.pallas_call(
        _fin_body,
        grid=(_GRID,),
        in_specs=[
            pl.BlockSpec((2, 512, FEAT), lambda i: (0, i, 0)),
            _bs2(512, FEAT, lambda i: (i, 0)),
            _bs2(512, 1, lambda i: (i, 0)),
            _bs2(512, 1, lambda i: (i, 0)),
            _bs2(1, FEAT, lambda i: (0, 0)),
            _bs2(512, FEAT, lambda i: (i, 0)),
            _bs2(512, FEAT, lambda i: (i, 0)),
            _bs2(FEAT, FEAT, lambda i: (0, 0)),
            _bs2(1, FEAT, lambda i: (0, 0)),
        ],
        out_specs=_bs2(512, FEAT, lambda i: (i, 0)),
        out_shape=jax.ShapeDtypeStruct((NP, FEAT), jnp.float32),
    )(sp, h, dis, inv, b1r, x1, x2, Wl, blr)


def kernel(utter_hidden, edge_index, edge_weight, posemb, W1, b1, Wl, bl):
    turn, batch, _ = utter_hidden.shape
    n = turn * batch
    x = jnp.transpose(utter_hidden, (1, 0, 2)).reshape(n, -1)
    pe = jnp.tile(posemb[:turn], (batch, 1))
    x = jnp.concatenate([x, pe], axis=1)
    x = jnp.zeros((NP, FEAT), jnp.float32).at[:n].set(x)

    row4 = edge_index[0].reshape(NW, NBLK, BLK, CH)
    col4 = edge_index[1].reshape(NW, NBLK, BLK, CH)
    ew4 = edge_weight.reshape(NW, NBLK, BLK, CH)
    col2 = edge_index[1].reshape(NW, CHUNKS, CH)
    ew2 = edge_weight.reshape(NW, CHUNKS, CH)
    b1r = b1.reshape(1, -1)
    blr = bl.reshape(1, -1)

    parts = _deg_kernel(col2, ew2)
    h1, g1, dis, inv = _tc_prep(parts, x, W1)
    s1 = _spmm_kernel(g1, row4, col4, ew4)
    x1, h2, g2 = _tc_mid(s1, h1, dis, inv, b1r, W1)
    s2 = _spmm_kernel(g2, row4, col4, ew4)
    x2, h3, g3 = _tc_mid(s2, h2, dis, inv, b1r, W1)
    s3 = _spmm_kernel(g3, row4, col4, ew4)
    y = _tc_fin(s3, h3, dis, inv, b1r, x1, x2, Wl, blr)
    return y[:n].reshape(batch, turn, -1)


# speedup vs baseline: 19.6978x; 1.1246x over previous
import functools

import jax
import jax.numpy as jnp
from jax import lax
from jax.experimental import pallas as pl
from jax.experimental.pallas import tpu as pltpu
from jax.experimental.pallas import tpu_sc as plsc

FEAT = 128
N_EDGES = 320000
NP = 10240
CH = 80
SG = CH // 16

_info = plsc.get_sparse_core_info()
NC = _info.num_cores
NS = _info.num_subcores
NW = NC * NS
EPW = N_EDGES // NW
CHUNKS = EPW // CH
BLK = 25
NBLK = CHUNKS // BLK
RPT = NP // NS

_MESH = plsc.VectorSubcoreMesh(core_axis_name="c", subcore_axis_name="s")


@functools.partial(
    pl.kernel,
    mesh=_MESH,
    out_type=jax.ShapeDtypeStruct((NC, NP), jnp.float32),
    scratch_types=[
        pltpu.VMEM((CHUNKS, CH), jnp.int32),
        pltpu.VMEM((CHUNKS, CH), jnp.float32),
        pltpu.VMEM((RPT,), jnp.float32),
        pltpu.VMEM_SHARED((NP,), jnp.float32),
    ],
)
def _deg_kernel(col_hbm, ew_hbm, out_hbm, col_v, ew_v, zbuf, deg_sh):
    c = lax.axis_index("c")
    s = lax.axis_index("s")
    wid = s * NC + c
    zero16 = jnp.zeros((16,), jnp.float32)

    def zinit(j, _):
        zbuf[pl.ds(j * 16, 16)] = zero16
        return 0

    lax.fori_loop(0, RPT // 16, zinit, 0)
    pltpu.sync_copy(zbuf, deg_sh.at[pl.ds(s * RPT, RPT)])
    pltpu.sync_copy(col_hbm.at[wid], col_v)
    pltpu.sync_copy(ew_hbm.at[wid], ew_v)
    plsc.subcore_barrier()

    def chunk(i, _):
        pltpu.sync_copy(ew_v.at[i], deg_sh.at[col_v.at[i]], add=True)
        return 0

    lax.fori_loop(0, CHUNKS, chunk, 0)
    plsc.subcore_barrier()
    pltpu.sync_copy(deg_sh.at[pl.ds(s * RPT, RPT)],
                    out_hbm.at[c, pl.ds(s * RPT, RPT)])


@functools.partial(
    pl.kernel,
    mesh=_MESH,
    out_type=jax.ShapeDtypeStruct((NC, NP, FEAT), jnp.float32),
    scratch_types=[
        pltpu.VMEM((BLK, CH), jnp.int32),
        pltpu.VMEM((BLK, CH), jnp.int32),
        pltpu.VMEM((BLK, CH), jnp.float32),
        pltpu.VMEM((CH, FEAT), jnp.float32),
        pltpu.VMEM((CH, FEAT), jnp.float32),
        pltpu.VMEM_SHARED((NP, FEAT), jnp.float32),
        pltpu.SemaphoreType.DMA,
        pltpu.SemaphoreType.DMA,
        pltpu.SemaphoreType.DMA,
        pltpu.SemaphoreType.DMA,
    ],
)
def _spmm_kernel(g_hbm, row_hbm, col_hbm, ew_hbm, out_hbm,
                 row_v, col_v, ew_v, rows_a, rows_b, acc_sh,
                 semg0, semg1, sems_a, sems_b):
    c = lax.axis_index("c")
    s = lax.axis_index("s")
    wid = s * NC + c
    zero16 = jnp.zeros((16,), jnp.float32)

    def zinit(j, _):
        for d in range(FEAT // 16):
            rows_a[j, pl.ds(d * 16, 16)] = zero16
        return 0

    lax.fori_loop(0, CH, zinit, 0)
    for k in range(RPT // CH):
        pltpu.sync_copy(rows_a, acc_sh.at[pl.ds(s * RPT + k * CH, CH)])
    plsc.subcore_barrier()

    zidx = jnp.zeros((16,), jnp.int32)

    def do_chunk(buf, ssem, i):
        def grp(gi, _):
            wv = ew_v[i, pl.ds(gi * 16, 16)]
            cv = col_v[i, pl.ds(gi * 16, 16)]
            base = gi * 16
            for l in range(16):
                w = wv[l]
                j = base + l
                for d in range(FEAT // 16):
                    sl = pl.ds(d * 16, 16)
                    buf[j, sl] = buf[j, sl] * w
            pltpu.async_copy(buf.at[pl.ds(base, 16)],
                             acc_sh.at[cv], ssem, add=True)
            return 0

        lax.fori_loop(0, SG, grp, 0)

    def drain(buf, ssem):
        for _gi in range(SG):
            pltpu.make_async_copy(buf.at[pl.ds(0, 16)],
                                  acc_sh.at[zidx], ssem).wait()

    def blk_body(bi, _):
        pltpu.sync_copy(row_hbm.at[wid, bi], row_v)
        pltpu.sync_copy(col_hbm.at[wid, bi], col_v)
        pltpu.sync_copy(ew_hbm.at[wid, bi], ew_v)
        pltpu.async_copy(g_hbm.at[row_v.at[0]], rows_a, semg0)
        pltpu.async_copy(g_hbm.at[row_v.at[1]], rows_b, semg1)

        def pair(k, _):
            i0 = 2 * k
            i1 = 2 * k + 1
            i2 = 2 * k + 2
            i3 = 2 * k + 3
            pltpu.make_async_copy(g_hbm.at[row_v.at[i0]], rows_a, semg0).wait()
            do_chunk(rows_a, sems_a, i0)
            pltpu.make_async_copy(g_hbm.at[row_v.at[i1]], rows_b, semg1).wait()
            drain(rows_a, sems_a)
            pltpu.async_copy(g_hbm.at[row_v.at[i2]], rows_a, semg0)
            do_chunk(rows_b, sems_b, i1)
            drain(rows_b, sems_b)

            @pl.when(i3 < BLK)
            def _():
                pltpu.async_copy(g_hbm.at[row_v.at[i3]], rows_b, semg1)

            return 0

        lax.fori_loop(0, BLK // 2, pair, 0)
        tail = BLK - 1
        pltpu.make_async_copy(g_hbm.at[row_v.at[tail]], rows_a, semg0).wait()
        do_chunk(rows_a, sems_a, tail)
        drain(rows_a, sems_a)
        return 0

    lax.fori_loop(0, NBLK, blk_body, 0)
    plsc.subcore_barrier()
    pltpu.sync_copy(acc_sh.at[pl.ds(s * RPT, RPT)],
                    out_hbm.at[c, pl.ds(s * RPT, RPT)])


_GRID = NP // 512


def _bs2(r, c_, im):
    return pl.BlockSpec((r, c_), im)


def _prep_body(part_ref, x_ref, w_ref, g_ref, dis_ref):
    deg = part_ref[0, :] + part_ref[1, :] + 1.0
    dis = lax.rsqrt(deg)
    h = jnp.dot(x_ref[...], w_ref[...], preferred_element_type=jnp.float32)
    g_ref[...] = h * dis[:, None]
    dis_ref[...] = dis[:, None]


def _tc_prep(parts, x, W1):
    return pl.pallas_call(
        _prep_body,
        grid=(_GRID,),
        in_specs=[
            _bs2(2, 512, lambda i: (0, i)),
            _bs2(512, FEAT, lambda i: (i, 0)),
            _bs2(FEAT, FEAT, lambda i: (0, 0)),
        ],
        out_specs=[
            _bs2(512, FEAT, lambda i: (i, 0)),
            _bs2(512, 1, lambda i: (i, 0)),
        ],
        out_shape=[
            jax.ShapeDtypeStruct((NP, FEAT), jnp.float32),
            jax.ShapeDtypeStruct((NP, 1), jnp.float32),
        ],
    )(parts, x, W1)


def _mid_body(sp_ref, g_ref, dis_ref, b1_ref, w_ref, x_ref, gn_ref):
    sacc = sp_ref[0] + sp_ref[1] + g_ref[...]
    xl = jnp.maximum(dis_ref[...] * sacc + b1_ref[...], 0.0)
    x_ref[...] = xl
    hn = jnp.dot(xl, w_ref[...], preferred_element_type=jnp.float32)
    gn_ref[...] = dis_ref[...] * hn


def _tc_mid(sp, g, dis, b1r, W1):
    return pl.pallas_call(
        _mid_body,
        grid=(_GRID,),
        in_specs=[
            pl.BlockSpec((2, 512, FEAT), lambda i: (0, i, 0)),
            _bs2(512, FEAT, lambda i: (i, 0)),
            _bs2(512, 1, lambda i: (i, 0)),
            _bs2(1, FEAT, lambda i: (0, 0)),
            _bs2(FEAT, FEAT, lambda i: (0, 0)),
        ],
        out_specs=[
            _bs2(512, FEAT, lambda i: (i, 0)),
            _bs2(512, FEAT, lambda i: (i, 0)),
        ],
        out_shape=[
            jax.ShapeDtypeStruct((NP, FEAT), jnp.float32),
            jax.ShapeDtypeStruct((NP, FEAT), jnp.float32),
        ],
    )(sp, g, dis, b1r, W1)


def _fin_body(sp_ref, g_ref, dis_ref, b1_ref, x1_ref, x2_ref,
              wl_ref, bl_ref, y_ref):
    sacc = sp_ref[0] + sp_ref[1] + g_ref[...]
    x3 = jnp.maximum(dis_ref[...] * sacc + b1_ref[...], 0.0)
    xs = x1_ref[...] + x2_ref[...] + x3
    y_ref[...] = jnp.tanh(
        jnp.dot(xs, wl_ref[...], preferred_element_type=jnp.float32)
        + bl_ref[...])


def _tc_fin(sp, g, dis, b1r, x1, x2, Wl, blr):
    return pl.pallas_call(
        _fin_body,
        grid=(_GRID,),
        in_specs=[
            pl.BlockSpec((2, 512, FEAT), lambda i: (0, i, 0)),
            _bs2(512, FEAT, lambda i: (i, 0)),
            _bs2(512, 1, lambda i: (i, 0)),
            _bs2(1, FEAT, lambda i: (0, 0)),
            _bs2(512, FEAT, lambda i: (i, 0)),
            _bs2(512, FEAT, lambda i: (i, 0)),
            _bs2(FEAT, FEAT, lambda i: (0, 0)),
            _bs2(1, FEAT, lambda i: (0, 0)),
        ],
        out_specs=_bs2(512, FEAT, lambda i: (i, 0)),
        out_shape=jax.ShapeDtypeStruct((NP, FEAT), jnp.float32),
    )(sp, g, dis, b1r, x1, x2, Wl, blr)


def kernel(utter_hidden, edge_index, edge_weight, posemb, W1, b1, Wl, bl):
    turn, batch, _ = utter_hidden.shape
    n = turn * batch
    x = jnp.transpose(utter_hidden, (1, 0, 2)).reshape(n, -1)
    pe = jnp.tile(posemb[:turn], (batch, 1))
    x = jnp.concatenate([x, pe], axis=1)
    x = jnp.zeros((NP, FEAT), jnp.float32).at[:n].set(x)

    row4 = edge_index[0].reshape(NW, NBLK, BLK, CH)
    col4 = edge_index[1].reshape(NW, NBLK, BLK, CH)
    ew4 = edge_weight.reshape(NW, NBLK, BLK, CH)
    col2 = edge_index[1].reshape(NW, CHUNKS, CH)
    ew2 = edge_weight.reshape(NW, CHUNKS, CH)
    b1r = b1.reshape(1, -1)
    blr = bl.reshape(1, -1)

    parts = _deg_kernel(col2, ew2)
    g1, dis = _tc_prep(parts, x, W1)
    s1 = _spmm_kernel(g1, row4, col4, ew4)
    x1, g2 = _tc_mid(s1, g1, dis, b1r, W1)
    s2 = _spmm_kernel(g2, row4, col4, ew4)
    x2, g3 = _tc_mid(s2, g2, dis, b1r, W1)
    s3 = _spmm_kernel(g3, row4, col4, ew4)
    y = _tc_fin(s3, g3, dis, b1r, x1, x2, Wl, blr)
    return y[:n].reshape(batch, turn, -1)


# speedup vs baseline: 20.2583x; 1.0285x over previous
import functools

import jax
import jax.numpy as jnp
from jax import lax
from jax.experimental import pallas as pl
from jax.experimental.pallas import tpu as pltpu
from jax.experimental.pallas import tpu_sc as plsc

FEAT = 128
N_EDGES = 320000
NP = 10240
CH = 80
SG = CH // 16

_info = plsc.get_sparse_core_info()
NC = _info.num_cores
NS = _info.num_subcores
NW = NC * NS
EPW = N_EDGES // NW
CHUNKS = EPW // CH
BLK = 25
NBLK = CHUNKS // BLK
RPT = NP // NS

_MESH = plsc.VectorSubcoreMesh(core_axis_name="c", subcore_axis_name="s")


@functools.partial(
    pl.kernel,
    mesh=_MESH,
    out_type=jax.ShapeDtypeStruct((NC, NP), jnp.float32),
    scratch_types=[
        pltpu.VMEM((CHUNKS, CH), jnp.int32),
        pltpu.VMEM((CHUNKS, CH), jnp.float32),
        pltpu.VMEM((RPT,), jnp.float32),
        pltpu.VMEM_SHARED((NP,), jnp.float32),
    ],
)
def _deg_kernel(col_hbm, ew_hbm, out_hbm, col_v, ew_v, zbuf, deg_sh):
    c = lax.axis_index("c")
    s = lax.axis_index("s")
    wid = s * NC + c
    zero16 = jnp.zeros((16,), jnp.float32)

    def zinit(j, _):
        zbuf[pl.ds(j * 16, 16)] = zero16
        return 0

    lax.fori_loop(0, RPT // 16, zinit, 0)
    pltpu.sync_copy(zbuf, deg_sh.at[pl.ds(s * RPT, RPT)])
    pltpu.sync_copy(col_hbm.at[wid], col_v)
    pltpu.sync_copy(ew_hbm.at[wid], ew_v)
    plsc.subcore_barrier()

    def chunk(i, _):
        pltpu.sync_copy(ew_v.at[i], deg_sh.at[col_v.at[i]], add=True)
        return 0

    lax.fori_loop(0, CHUNKS, chunk, 0)
    plsc.subcore_barrier()
    pltpu.sync_copy(deg_sh.at[pl.ds(s * RPT, RPT)],
                    out_hbm.at[c, pl.ds(s * RPT, RPT)])


@functools.partial(
    pl.kernel,
    mesh=_MESH,
    out_type=jax.ShapeDtypeStruct((NC, NP, FEAT), jnp.float32),
    scratch_types=[
        pltpu.VMEM((BLK, CH), jnp.int32),
        pltpu.VMEM((BLK, CH), jnp.int32),
        pltpu.VMEM((BLK, CH), jnp.float32),
        pltpu.VMEM((CH, FEAT), jnp.float32),
        pltpu.VMEM((CH, FEAT), jnp.float32),
        pltpu.VMEM_SHARED((NP, FEAT), jnp.float32),
        pltpu.SemaphoreType.DMA,
        pltpu.SemaphoreType.DMA,
        pltpu.SemaphoreType.DMA,
        pltpu.SemaphoreType.DMA,
        pltpu.SemaphoreType.DMA,
        pltpu.SemaphoreType.DMA,
    ],
)
def _spmm_kernel(g_hbm, row_hbm, col_hbm, ew_hbm, out_hbm,
                 row_v, col_v, ew_v, rows_a, rows_b, acc_sh,
                 ga_lo, ga_hi, gb_lo, gb_hi, sems_a, sems_b):
    c = lax.axis_index("c")
    s = lax.axis_index("s")
    wid = s * NC + c
    zero16 = jnp.zeros((16,), jnp.float32)

    def zinit(j, _):
        for d in range(FEAT // 16):
            rows_a[j, pl.ds(d * 16, 16)] = zero16
        return 0

    lax.fori_loop(0, CH, zinit, 0)
    for k in range(RPT // CH):
        pltpu.sync_copy(rows_a, acc_sh.at[pl.ds(s * RPT + k * CH, CH)])
    plsc.subcore_barrier()

    zidx = jnp.zeros((16,), jnp.int32)

    LO = 48

    def grp_body(buf, ssem, i, gi):
        wv = ew_v[i, pl.ds(gi * 16, 16)]
        cv = col_v[i, pl.ds(gi * 16, 16)]
        base = gi * 16
        for l in range(16):
            w = wv[l]
            j = base + l
            for d in range(FEAT // 16):
                sl = pl.ds(d * 16, 16)
                buf[j, sl] = buf[j, sl] * w
        pltpu.async_copy(buf.at[pl.ds(base, 16)],
                         acc_sh.at[cv], ssem, add=True)

    def do_lo(buf, ssem, i):
        def grp(gi, _):
            grp_body(buf, ssem, i, gi)
            return 0
        lax.fori_loop(0, LO // 16, grp, 0)

    def do_hi(buf, ssem, i):
        def grp(gi, _):
            grp_body(buf, ssem, i, gi)
            return 0
        lax.fori_loop(LO // 16, SG, grp, 0)

    def start_gathers(buf, slo, shi, i):
        pltpu.async_copy(g_hbm.at[row_v.at[i, pl.ds(0, LO)]],
                         buf.at[pl.ds(0, LO)], slo)
        pltpu.async_copy(g_hbm.at[row_v.at[i, pl.ds(LO, CH - LO)]],
                         buf.at[pl.ds(LO, CH - LO)], shi)

    def wait_lo(buf, slo, i):
        pltpu.make_async_copy(g_hbm.at[row_v.at[i, pl.ds(0, LO)]],
                              buf.at[pl.ds(0, LO)], slo).wait()

    def wait_hi(buf, shi, i):
        pltpu.make_async_copy(g_hbm.at[row_v.at[i, pl.ds(LO, CH - LO)]],
                              buf.at[pl.ds(LO, CH - LO)], shi).wait()

    def drain(buf, ssem):
        for _gi in range(SG):
            pltpu.make_async_copy(buf.at[pl.ds(0, 16)],
                                  acc_sh.at[zidx], ssem).wait()

    def blk_body(bi, _):
        pltpu.sync_copy(row_hbm.at[wid, bi], row_v)
        pltpu.sync_copy(col_hbm.at[wid, bi], col_v)
        pltpu.sync_copy(ew_hbm.at[wid, bi], ew_v)
        start_gathers(rows_a, ga_lo, ga_hi, 0)
        start_gathers(rows_b, gb_lo, gb_hi, 1)

        def pair(k, _):
            i0 = 2 * k
            i1 = 2 * k + 1
            i2 = 2 * k + 2
            i3 = 2 * k + 3
            wait_lo(rows_a, ga_lo, i0)
            do_lo(rows_a, sems_a, i0)
            wait_hi(rows_a, ga_hi, i0)
            do_hi(rows_a, sems_a, i0)
            drain(rows_a, sems_a)
            start_gathers(rows_a, ga_lo, ga_hi, i2)
            wait_lo(rows_b, gb_lo, i1)
            do_lo(rows_b, sems_b, i1)
            wait_hi(rows_b, gb_hi, i1)
            do_hi(rows_b, sems_b, i1)
            drain(rows_b, sems_b)

            @pl.when(i3 < BLK)
            def _():
                start_gathers(rows_b, gb_lo, gb_hi, i3)

            return 0

        lax.fori_loop(0, BLK // 2, pair, 0)
        tail = BLK - 1
        wait_lo(rows_a, ga_lo, tail)
        do_lo(rows_a, sems_a, tail)
        wait_hi(rows_a, ga_hi, tail)
        do_hi(rows_a, sems_a, tail)
        drain(rows_a, sems_a)
        return 0

    lax.fori_loop(0, NBLK, blk_body, 0)
    plsc.subcore_barrier()
    pltpu.sync_copy(acc_sh.at[pl.ds(s * RPT, RPT)],
                    out_hbm.at[c, pl.ds(s * RPT, RPT)])


_GRID = NP // 512


def _bs2(r, c_, im):
    return pl.BlockSpec((r, c_), im)


def _prep_body(part_ref, x_ref, w_ref, g_ref, dis_ref):
    deg = part_ref[0, :] + part_ref[1, :] + 1.0
    dis = lax.rsqrt(deg)
    h = jnp.dot(x_ref[...], w_ref[...], preferred_element_type=jnp.float32)
    g_ref[...] = h * dis[:, None]
    dis_ref[...] = dis[:, None]


def _tc_prep(parts, x, W1):
    return pl.pallas_call(
        _prep_body,
        grid=(_GRID,),
        in_specs=[
            _bs2(2, 512, lambda i: (0, i)),
            _bs2(512, FEAT, lambda i: (i, 0)),
            _bs2(FEAT, FEAT, lambda i: (0, 0)),
        ],
        out_specs=[
            _bs2(512, FEAT, lambda i: (i, 0)),
            _bs2(512, 1, lambda i: (i, 0)),
        ],
        out_shape=[
            jax.ShapeDtypeStruct((NP, FEAT), jnp.float32),
            jax.ShapeDtypeStruct((NP, 1), jnp.float32),
        ],
    )(parts, x, W1)


def _mid_body(sp_ref, g_ref, dis_ref, b1_ref, w_ref, x_ref, gn_ref):
    sacc = sp_ref[0] + sp_ref[1] + g_ref[...]
    xl = jnp.maximum(dis_ref[...] * sacc + b1_ref[...], 0.0)
    x_ref[...] = xl
    hn = jnp.dot(xl, w_ref[...], preferred_element_type=jnp.float32)
    gn_ref[...] = dis_ref[...] * hn


def _tc_mid(sp, g, dis, b1r, W1):
    return pl.pallas_call(
        _mid_body,
        grid=(_GRID,),
        in_specs=[
            pl.BlockSpec((2, 512, FEAT), lambda i: (0, i, 0)),
            _bs2(512, FEAT, lambda i: (i, 0)),
            _bs2(512, 1, lambda i: (i, 0)),
            _bs2(1, FEAT, lambda i: (0, 0)),
            _bs2(FEAT, FEAT, lambda i: (0, 0)),
        ],
        out_specs=[
            _bs2(512, FEAT, lambda i: (i, 0)),
            _bs2(512, FEAT, lambda i: (i, 0)),
        ],
        out_shape=[
            jax.ShapeDtypeStruct((NP, FEAT), jnp.float32),
            jax.ShapeDtypeStruct((NP, FEAT), jnp.float32),
        ],
    )(sp, g, dis, b1r, W1)


def _fin_body(sp_ref, g_ref, dis_ref, b1_ref, x1_ref, x2_ref,
              wl_ref, bl_ref, y_ref):
    sacc = sp_ref[0] + sp_ref[1] + g_ref[...]
    x3 = jnp.maximum(dis_ref[...] * sacc + b1_ref[...], 0.0)
    xs = x1_ref[...] + x2_ref[...] + x3
    y_ref[...] = jnp.tanh(
        jnp.dot(xs, wl_ref[...], preferred_element_type=jnp.float32)
        + bl_ref[...])


def _tc_fin(sp, g, dis, b1r, x1, x2, Wl, blr):
    return pl.pallas_call(
        _fin_body,
        grid=(_GRID,),
        in_specs=[
            pl.BlockSpec((2, 512, FEAT), lambda i: (0, i, 0)),
            _bs2(512, FEAT, lambda i: (i, 0)),
            _bs2(512, 1, lambda i: (i, 0)),
            _bs2(1, FEAT, lambda i: (0, 0)),
            _bs2(512, FEAT, lambda i: (i, 0)),
            _bs2(512, FEAT, lambda i: (i, 0)),
            _bs2(FEAT, FEAT, lambda i: (0, 0)),
            _bs2(1, FEAT, lambda i: (0, 0)),
        ],
        out_specs=_bs2(512, FEAT, lambda i: (i, 0)),
        out_shape=jax.ShapeDtypeStruct((NP, FEAT), jnp.float32),
    )(sp, g, dis, b1r, x1, x2, Wl, blr)


def kernel(utter_hidden, edge_index, edge_weight, posemb, W1, b1, Wl, bl):
    turn, batch, _ = utter_hidden.shape
    n = turn * batch
    x = jnp.transpose(utter_hidden, (1, 0, 2)).reshape(n, -1)
    pe = jnp.tile(posemb[:turn], (batch, 1))
    x = jnp.concatenate([x, pe], axis=1)
    x = jnp.zeros((NP, FEAT), jnp.float32).at[:n].set(x)

    row4 = edge_index[0].reshape(NW, NBLK, BLK, CH)
    col4 = edge_index[1].reshape(NW, NBLK, BLK, CH)
    ew4 = edge_weight.reshape(NW, NBLK, BLK, CH)
    col2 = edge_index[1].reshape(NW, CHUNKS, CH)
    ew2 = edge_weight.reshape(NW, CHUNKS, CH)
    b1r = b1.reshape(1, -1)
    blr = bl.reshape(1, -1)

    parts = _deg_kernel(col2, ew2)
    g1, dis = _tc_prep(parts, x, W1)
    s1 = _spmm_kernel(g1, row4, col4, ew4)
    x1, g2 = _tc_mid(s1, g1, dis, b1r, W1)
    s2 = _spmm_kernel(g2, row4, col4, ew4)
    x2, g3 = _tc_mid(s2, g2, dis, b1r, W1)
    s3 = _spmm_kernel(g3, row4, col4, ew4)
    y = _tc_fin(s3, g3, dis, b1r, x1, x2, Wl, blr)
    return y[:n].reshape(batch, turn, -1)
